# baseline (speedup 1.0000x reference)
import functools

import jax
import jax.numpy as jnp
import numpy as np
from jax import lax
from jax.experimental import pallas as pl
from jax.experimental.pallas import tpu as pltpu
from jax.experimental.pallas import tpu_sc as plsc

B = 16384
V = 1000000
D = 64
NEG = 5

NC = 2
NS = 16
L = 16
NW = NC * NS
C = B // NW
S = 128
NSUB = C // S
NT = D // L


def _sc_body(pu_h, pv_h, nv_h, emb_h, out_h, idxu, idxv, idxn, qu, qv, qn,
             ru, rv, rn, dots, sem):
    wid = lax.axis_index("s") * NC + lax.axis_index("c")
    pltpu.sync_copy(pu_h.at[wid], idxu)
    pltpu.sync_copy(pv_h.at[wid], idxv)
    pltpu.sync_copy(nv_h.at[wid], idxn)

    for r in range(NSUB):
        for t in range(S // L):
            sl = pl.ds(L * t, L)
            qu[r, sl] = jax.lax.shift_right_logical(idxu[r, sl], 1)
            qv[r, sl] = jax.lax.shift_right_logical(idxv[r, sl], 1)
    for r in range(NSUB * NEG):
        for t in range(S // L):
            sl = pl.ds(L * t, L)
            qn[r, sl] = jax.lax.shift_right_logical(idxn[r, sl], 1)

    for j in range(NSUB):
        cps = [
            pltpu.async_copy(emb_h.at[qu.at[j]], ru, sem),
            pltpu.async_copy(emb_h.at[qv.at[j]], rv, sem),
        ]
        for m in range(NEG):
            cps.append(
                pltpu.async_copy(emb_h.at[qn.at[j * NEG + m]],
                                 rn.at[pl.ds(m * S, S)], sem))
        for c in cps:
            c.wait()

        lane = lax.broadcasted_iota(jnp.int32, (L,), 0)

        def grp(g, carry, j=j):
            bidx = g * L + lane
            pu_i = idxu[j, pl.ds(g * L, L)]
            pv_i = idxv[j, pl.ds(g * L, L)]
            cu = (pu_i & 1) * D
            cv = (pv_i & 1) * D
            nrow, cn = [], []
            for k in range(NEG):
                f = bidx * NEG + k
                nidx = plsc.load_gather(idxn, [j * NEG + (f >> 7), f & (S - 1)])
                nrow.append(f)
                cn.append((nidx & 1) * D)
            def dstep(d, acc):
                u_d = plsc.load_gather(ru, [bidx, cu + d])
                v_d = plsc.load_gather(rv, [bidx, cv + d])
                acc0 = acc[0] + u_d * v_d
                rest = []
                for k in range(NEG):
                    n_d = plsc.load_gather(rn, [nrow[k], cn[k] + d])
                    rest.append(acc[1 + k] + u_d * n_d)
                return (acc0, *rest)

            acc = lax.fori_loop(0, D, dstep,
                                tuple(jnp.zeros((L,), jnp.float32)
                                      for _ in range(6)),
                                unroll=4)
            for jd in range(6):
                dots[jd, pl.ds(j * S + g * L, L)] = acc[jd]
            return carry

        lax.fori_loop(0, S // L, grp, 0)

    pltpu.sync_copy(dots, out_h.at[wid])


@functools.cache
def _make_sc_call():
    return functools.partial(
        pl.kernel,
        out_type=jax.ShapeDtypeStruct((NW, 6, C), jnp.float32),
        mesh=plsc.VectorSubcoreMesh(core_axis_name="c", subcore_axis_name="s"),
        compiler_params=pltpu.CompilerParams(needs_layout_passes=False),
        scratch_types=[
            pltpu.VMEM((NSUB, S), jnp.int32),
            pltpu.VMEM((NSUB, S), jnp.int32),
            pltpu.VMEM((NSUB * NEG, S), jnp.int32),
            pltpu.VMEM((NSUB, S), jnp.int32),
            pltpu.VMEM((NSUB, S), jnp.int32),
            pltpu.VMEM((NSUB * NEG, S), jnp.int32),
            pltpu.VMEM((S, 2 * D), jnp.float32),
            pltpu.VMEM((S, 2 * D), jnp.float32),
            pltpu.VMEM((S * NEG, 2 * D), jnp.float32),
            pltpu.VMEM((6, C), jnp.float32),
            pltpu.SemaphoreType.DMA,
        ],
    )(_sc_body)


def _tc_body(d_ref, o_ref):
    x = d_ref[...]
    x = jnp.clip(x, -10.0, 10.0)
    sgn = jnp.where(lax.broadcasted_iota(jnp.int32, (1, 6, 1), 1) == 0,
                    -1.0, 1.0).astype(jnp.float32)
    loss = jnp.log1p(jnp.exp(x * sgn))
    o_ref[0, 0] = jnp.sum(loss) / np.float32(B)


_tc_call = pl.pallas_call(
    _tc_body,
    out_shape=jax.ShapeDtypeStruct((1, 1), jnp.float32),
    out_specs=pl.BlockSpec(memory_space=pltpu.SMEM),
)


def kernel(pos_u, pos_v, neg_v, embeddings):
    pu = pos_u.astype(jnp.int32).reshape(NW, NSUB, S)
    pv = pos_v.astype(jnp.int32).reshape(NW, NSUB, S)
    nv = neg_v.astype(jnp.int32).reshape(NW, NSUB * NEG, S)
    emb2 = embeddings.reshape(V // 2, 2 * D)
    dots = _make_sc_call()(pu, pv, nv, emb2)
    return _tc_call(dots)[0, 0]


# speedup vs baseline: 1.3871x; 1.3871x over previous
import functools

import jax
import jax.numpy as jnp
import numpy as np
from jax import lax
from jax.experimental import pallas as pl
from jax.experimental.pallas import tpu as pltpu
from jax.experimental.pallas import tpu_sc as plsc

B = 16384
V = 1000000
D = 64
NEG = 5

NC = 2
NS = 16
L = 16
NW = NC * NS
C = B // NW
S = 128
NSUB = C // S
NT = D // L

TW = 2048
TG = -(-V // TW)


def _tp_body(a_ref, o_ref):
    o_ref[:, 0:D] = a_ref[...].T


_tp_call = pl.pallas_call(
    _tp_body,
    grid=(TG,),
    in_specs=[pl.BlockSpec((D, TW), lambda i: (0, i))],
    out_specs=pl.BlockSpec((TW, 2 * D), lambda i: (i, 0)),
    out_shape=jax.ShapeDtypeStruct((V, 2 * D), jnp.float32),
)


def _sc_body(pu_h, pv_h, nv_h, emb_h, out_h, idxu, idxv, idxn, ru, rv, rn,
             dots, sem):
    wid = lax.axis_index("s") * NC + lax.axis_index("c")
    pltpu.sync_copy(pu_h.at[wid], idxu)
    pltpu.sync_copy(pv_h.at[wid], idxv)
    pltpu.sync_copy(nv_h.at[wid], idxn)

    for j in range(NSUB):
        cps = [
            pltpu.async_copy(emb_h.at[idxu.at[j]], ru, sem),
            pltpu.async_copy(emb_h.at[idxv.at[j]], rv, sem),
        ]
        for m in range(NEG):
            cps.append(
                pltpu.async_copy(emb_h.at[idxn.at[j * NEG + m]],
                                 rn.at[pl.ds(m * S, S)], sem))
        for c in cps:
            c.wait()

        lane = lax.broadcasted_iota(jnp.int32, (L,), 0)

        def body(b, carry, j=j):
            u = [ru[b, pl.ds(L * t, L)] for t in range(NT)]
            v = [rv[b, pl.ds(L * t, L)] for t in range(NT)]
            mask = lane == (b & (L - 1))
            out = []
            acc = (u[0] * v[0] + u[1] * v[1]) + (u[2] * v[2] + u[3] * v[3])
            out.append(jnp.where(mask, jnp.sum(acc), carry[0]))
            for k in range(NEG):
                w = [rn[b * NEG + k, pl.ds(L * t, L)] for t in range(NT)]
                acc = (u[0] * w[0] + u[1] * w[1]) + (u[2] * w[2] + u[3] * w[3])
                out.append(jnp.where(mask, jnp.sum(acc), carry[1 + k]))

            @pl.when((b & (L - 1)) == (L - 1))
            def _():
                for jd in range(6):
                    dots[jd, pl.ds(j * S + b - (L - 1), L)] = out[jd]

            return tuple(out)

        zero = jnp.zeros((L,), jnp.float32)
        lax.fori_loop(0, S, body, (zero,) * 6)

    pltpu.sync_copy(dots, out_h.at[wid])


@functools.cache
def _make_sc_call():
    return functools.partial(
        pl.kernel,
        out_type=jax.ShapeDtypeStruct((NW, 6, C), jnp.float32),
        mesh=plsc.VectorSubcoreMesh(core_axis_name="c", subcore_axis_name="s"),
        compiler_params=pltpu.CompilerParams(needs_layout_passes=False),
        scratch_types=[
            pltpu.VMEM((NSUB, S), jnp.int32),
            pltpu.VMEM((NSUB, S), jnp.int32),
            pltpu.VMEM((NSUB * NEG, S), jnp.int32),
            pltpu.VMEM((S, 2 * D), jnp.float32),
            pltpu.VMEM((S, 2 * D), jnp.float32),
            pltpu.VMEM((S * NEG, 2 * D), jnp.float32),
            pltpu.VMEM((6, C), jnp.float32),
            pltpu.SemaphoreType.DMA,
        ],
    )(_sc_body)


def _tc_body(d_ref, o_ref):
    x = d_ref[...]
    x = jnp.clip(x, -10.0, 10.0)
    sgn = jnp.where(lax.broadcasted_iota(jnp.int32, (1, 6, 1), 1) == 0,
                    -1.0, 1.0).astype(jnp.float32)
    loss = jnp.log1p(jnp.exp(x * sgn))
    o_ref[0, 0] = jnp.sum(loss) / np.float32(B)


_tc_call = pl.pallas_call(
    _tc_body,
    out_shape=jax.ShapeDtypeStruct((1, 1), jnp.float32),
    out_specs=pl.BlockSpec(memory_space=pltpu.SMEM),
)


def kernel(pos_u, pos_v, neg_v, embeddings):
    pu = pos_u.astype(jnp.int32).reshape(NW, NSUB, S)
    pv = pos_v.astype(jnp.int32).reshape(NW, NSUB, S)
    nv = neg_v.astype(jnp.int32).reshape(NW, NSUB * NEG, S)
    emb2 = _tp_call(embeddings.T)
    dots = _make_sc_call()(pu, pv, nv, emb2)
    return _tc_call(dots)[0, 0]
